# baseline (speedup 1.0000x reference)
import jax
import jax.numpy as jnp
from jax.experimental import pallas as pl
from jax.experimental.pallas import tpu as pltpu

_VCHUNK = 2048


def _pointer_body(tok_ref, emb_ref, gen_ref, enc_ref, dec_ref, ah_ref,
                  w_ref, scal_ref, final_ref, ptr_ref, pg_ref):
    hp = ah_ref.shape[1]
    seq_i = ah_ref.shape[3]
    dm = enc_ref.shape[2]
    vocab = gen_ref.shape[2]

    attn = jnp.mean(ah_ref[0], axis=0)
    context = jax.lax.dot_general(
        attn, enc_ref[0], (((1,), (0,)), ((), ())),
        precision=jax.lax.Precision.HIGHEST,
        preferred_element_type=jnp.float32)

    w1 = w_ref[0:1, 0:dm]
    w2 = w_ref[0:1, dm:2 * dm]
    w3 = w_ref[0:1, 2 * dm:3 * dm]
    z = (jnp.sum(context * w1, axis=1, keepdims=True)
         + jnp.sum(dec_ref[0] * w2, axis=1, keepdims=True)
         + jnp.sum(emb_ref[0] * w3, axis=1, keepdims=True)
         + scal_ref[0])
    p_gen = jax.nn.sigmoid(z)
    sw = scal_ref[1]
    sb = scal_ref[2]

    tok = tok_ref[0]
    for k0 in range(0, vocab, _VCHUNK):
        ck = min(_VCHUNK, vocab - k0)
        iota = jax.lax.broadcasted_iota(jnp.int32, (seq_i, ck), 1) + k0
        oh = (tok == iota).astype(jnp.float32)
        xk = jax.lax.dot_general(
            attn, oh, (((1,), (0,)), ((), ())),
            precision=jax.lax.Precision.HIGHEST,
            preferred_element_type=jnp.float32)
        ptr_ref[0, :, k0:k0 + ck] = xk

    x = ptr_ref[0]
    m = jnp.max(x, axis=1, keepdims=True)
    lse = m + jnp.log(jnp.sum(jnp.exp(x - m), axis=1, keepdims=True))
    ptr = sw * (x - lse) + sb
    ptr_ref[0] = ptr
    final_ref[0] = p_gen * gen_ref[0] + (1.0 - p_gen) * ptr
    pg_ref[0] = p_gen


def kernel(inp_tokens, tar_embedded, generator_output, enc_output,
           dec_state, attn_heads, W_pgen, b_pgen, scale_w, scale_b):
    b, t, vocab = generator_output.shape
    _, h, _, seq_i = attn_heads.shape
    dm = enc_output.shape[-1]
    tb = t // 2

    tok = inp_tokens.astype(jnp.int32).reshape(b, seq_i, 1)
    w_row = W_pgen.reshape(1, 3 * dm)
    scal = jnp.concatenate([
        jnp.reshape(b_pgen, (1,)), jnp.reshape(scale_w, (1,)),
        jnp.reshape(scale_b, (1,))
    ]).astype(jnp.float32)

    final, ptr, pg = pl.pallas_call(
        _pointer_body,
        grid=(b, t // tb),
        in_specs=[
            pl.BlockSpec((1, seq_i, 1), lambda i, j: (i, 0, 0)),
            pl.BlockSpec((1, tb, dm), lambda i, j: (i, j, 0)),
            pl.BlockSpec((1, tb, vocab), lambda i, j: (i, j, 0)),
            pl.BlockSpec((1, seq_i, dm), lambda i, j: (i, 0, 0)),
            pl.BlockSpec((1, tb, dm), lambda i, j: (i, j, 0)),
            pl.BlockSpec((1, h, tb, seq_i), lambda i, j: (i, 0, j, 0)),
            pl.BlockSpec((1, 3 * dm), lambda i, j: (0, 0)),
            pl.BlockSpec(memory_space=pltpu.SMEM),
        ],
        out_specs=[
            pl.BlockSpec((1, tb, vocab), lambda i, j: (i, j, 0)),
            pl.BlockSpec((1, tb, vocab), lambda i, j: (i, j, 0)),
            pl.BlockSpec((1, tb, 1), lambda i, j: (i, j, 0)),
        ],
        out_shape=[
            jax.ShapeDtypeStruct((b, t, vocab), jnp.float32),
            jax.ShapeDtypeStruct((b, t, vocab), jnp.float32),
            jax.ShapeDtypeStruct((b, t, 1), jnp.float32),
        ],
        compiler_params=pltpu.CompilerParams(
            dimension_semantics=("parallel", "parallel")),
    )(tok, tar_embedded, generator_output, enc_output, dec_state,
      attn_heads, w_row, scal)
    return final, ptr, pg[..., 0]


# speedup vs baseline: 1.8576x; 1.8576x over previous
import jax
import jax.numpy as jnp
from jax.experimental import pallas as pl
from jax.experimental.pallas import tpu as pltpu

_VCHUNK = 2048


def _pointer_body(tok_ref, emb_ref, gen_ref, enc_ref, dec_ref, ah_ref,
                  w_ref, scal_ref, final_ref, ptr_ref, pg_ref):
    hp = ah_ref.shape[1]
    seq_i = ah_ref.shape[3]
    dm = enc_ref.shape[2]
    vocab = gen_ref.shape[2]

    attn = jnp.mean(ah_ref[0], axis=0)
    context = jax.lax.dot_general(
        attn, enc_ref[0], (((1,), (0,)), ((), ())),
        precision=jax.lax.Precision.HIGHEST,
        preferred_element_type=jnp.float32)

    w1 = w_ref[0:1, 0:dm]
    w2 = w_ref[0:1, dm:2 * dm]
    w3 = w_ref[0:1, 2 * dm:3 * dm]
    z = (jnp.sum(context * w1, axis=1, keepdims=True)
         + jnp.sum(dec_ref[0] * w2, axis=1, keepdims=True)
         + jnp.sum(emb_ref[0] * w3, axis=1, keepdims=True)
         + scal_ref[0])
    p_gen = jax.nn.sigmoid(z)
    sw = scal_ref[1]
    sb = scal_ref[2]

    tok = tok_ref[0]
    attn_hi = attn.astype(jnp.bfloat16)
    attn_lo = (attn - attn_hi.astype(jnp.float32)).astype(jnp.bfloat16)
    for k0 in range(0, vocab, _VCHUNK):
        ck = min(_VCHUNK, vocab - k0)
        iota = jax.lax.broadcasted_iota(jnp.int32, (seq_i, ck), 1) + k0
        oh = (tok == iota).astype(jnp.bfloat16)
        xk = jax.lax.dot_general(
            attn_hi, oh, (((1,), (0,)), ((), ())),
            preferred_element_type=jnp.float32)
        xk += jax.lax.dot_general(
            attn_lo, oh, (((1,), (0,)), ((), ())),
            preferred_element_type=jnp.float32)
        ptr_ref[0, :, k0:k0 + ck] = xk

    x = ptr_ref[0]
    m = jnp.max(x, axis=1, keepdims=True)
    lse = m + jnp.log(jnp.sum(jnp.exp(x - m), axis=1, keepdims=True))
    ptr = sw * (x - lse) + sb
    ptr_ref[0] = ptr
    final_ref[0] = p_gen * gen_ref[0] + (1.0 - p_gen) * ptr
    pg_ref[0] = p_gen


def kernel(inp_tokens, tar_embedded, generator_output, enc_output,
           dec_state, attn_heads, W_pgen, b_pgen, scale_w, scale_b):
    b, t, vocab = generator_output.shape
    _, h, _, seq_i = attn_heads.shape
    dm = enc_output.shape[-1]
    tb = t

    tok = inp_tokens.astype(jnp.int32).reshape(b, seq_i, 1)
    w_row = W_pgen.reshape(1, 3 * dm)
    scal = jnp.concatenate([
        jnp.reshape(b_pgen, (1,)), jnp.reshape(scale_w, (1,)),
        jnp.reshape(scale_b, (1,))
    ]).astype(jnp.float32)

    final, ptr, pg = pl.pallas_call(
        _pointer_body,
        grid=(b, t // tb),
        in_specs=[
            pl.BlockSpec((1, seq_i, 1), lambda i, j: (i, 0, 0)),
            pl.BlockSpec((1, tb, dm), lambda i, j: (i, j, 0)),
            pl.BlockSpec((1, tb, vocab), lambda i, j: (i, j, 0)),
            pl.BlockSpec((1, seq_i, dm), lambda i, j: (i, 0, 0)),
            pl.BlockSpec((1, tb, dm), lambda i, j: (i, j, 0)),
            pl.BlockSpec((1, h, tb, seq_i), lambda i, j: (i, 0, j, 0)),
            pl.BlockSpec((1, 3 * dm), lambda i, j: (0, 0)),
            pl.BlockSpec(memory_space=pltpu.SMEM),
        ],
        out_specs=[
            pl.BlockSpec((1, tb, vocab), lambda i, j: (i, j, 0)),
            pl.BlockSpec((1, tb, vocab), lambda i, j: (i, j, 0)),
            pl.BlockSpec((1, tb, 1), lambda i, j: (i, j, 0)),
        ],
        out_shape=[
            jax.ShapeDtypeStruct((b, t, vocab), jnp.float32),
            jax.ShapeDtypeStruct((b, t, vocab), jnp.float32),
            jax.ShapeDtypeStruct((b, t, 1), jnp.float32),
        ],
        compiler_params=pltpu.CompilerParams(
            dimension_semantics=("parallel", "parallel"),
            vmem_limit_bytes=100 * 1024 * 1024),
    )(tok, tar_embedded, generator_output, enc_output, dec_state,
      attn_heads, w_row, scal)
    return final, ptr, pg[..., 0]


# speedup vs baseline: 3.0859x; 1.6612x over previous
import jax
import jax.numpy as jnp
from jax.experimental import pallas as pl
from jax.experimental.pallas import tpu as pltpu

_VCHUNK = 2000


def _pointer_body(tok_ref, emb_ref, genT_ref, enc_ref, dec_ref, ah_ref,
                  w_ref, scal_ref, finalT_ref, ptrT_ref, pg_ref):
    seq_i = ah_ref.shape[3]
    dm = enc_ref.shape[2]
    vocab = genT_ref.shape[1]

    attn = jnp.mean(ah_ref[0], axis=0)

    contextT = jax.lax.dot_general(
        enc_ref[0], attn, (((0,), (1,)), ((), ())),
        precision=jax.lax.Precision.HIGHEST,
        preferred_element_type=jnp.float32)

    w1 = w_ref[0:1, 0:dm]
    w2 = w_ref[0:1, dm:2 * dm]
    w3 = w_ref[0:1, 2 * dm:3 * dm]
    z = (jax.lax.dot_general(w1, contextT, (((1,), (0,)), ((), ())),
                             precision=jax.lax.Precision.HIGHEST,
                             preferred_element_type=jnp.float32)
         + jax.lax.dot_general(w2, dec_ref[0], (((1,), (1,)), ((), ())),
                               precision=jax.lax.Precision.HIGHEST,
                               preferred_element_type=jnp.float32)
         + jax.lax.dot_general(w3, emb_ref[0], (((1,), (1,)), ((), ())),
                               precision=jax.lax.Precision.HIGHEST,
                               preferred_element_type=jnp.float32)
         + scal_ref[0])
    p_gen = jax.nn.sigmoid(z)
    sw = scal_ref[1]
    sb = scal_ref[2]

    tok = tok_ref[0]
    attn_hi = attn.astype(jnp.bfloat16)
    attn_lo = (attn - attn_hi.astype(jnp.float32)).astype(jnp.bfloat16)
    for k0 in range(0, vocab, _VCHUNK):
        ck = min(_VCHUNK, vocab - k0)
        iota = jax.lax.broadcasted_iota(jnp.int32, (ck, seq_i), 0) + k0
        oh = (iota == tok).astype(jnp.bfloat16)
        xk = jax.lax.dot_general(
            oh, attn_hi, (((1,), (1,)), ((), ())),
            preferred_element_type=jnp.float32)
        xk += jax.lax.dot_general(
            oh, attn_lo, (((1,), (1,)), ((), ())),
            preferred_element_type=jnp.float32)
        ptrT_ref[0, k0:k0 + ck, :] = xk

    x = ptrT_ref[0]
    m = jnp.max(x, axis=0, keepdims=True)
    lse = m + jnp.log(jnp.sum(jnp.exp(x - m), axis=0, keepdims=True))
    ptr = sw * (x - lse) + sb
    ptrT_ref[0] = ptr
    finalT_ref[0] = p_gen * genT_ref[0] + (1.0 - p_gen) * ptr
    pg_ref[0] = p_gen


def kernel(inp_tokens, tar_embedded, generator_output, enc_output,
           dec_state, attn_heads, W_pgen, b_pgen, scale_w, scale_b):
    b, t, vocab = generator_output.shape
    _, h, _, seq_i = attn_heads.shape
    dm = enc_output.shape[-1]

    tok = inp_tokens.astype(jnp.int32).reshape(b, 1, seq_i)
    gen_t = jnp.transpose(generator_output, (0, 2, 1))
    w_row = W_pgen.reshape(1, 3 * dm)
    scal = jnp.concatenate([
        jnp.reshape(b_pgen, (1,)), jnp.reshape(scale_w, (1,)),
        jnp.reshape(scale_b, (1,))
    ]).astype(jnp.float32)

    final_t, ptr_t, pg = pl.pallas_call(
        _pointer_body,
        grid=(b,),
        in_specs=[
            pl.BlockSpec((1, 1, seq_i), lambda i: (i, 0, 0)),
            pl.BlockSpec((1, t, dm), lambda i: (i, 0, 0)),
            pl.BlockSpec((1, vocab, t), lambda i: (i, 0, 0)),
            pl.BlockSpec((1, seq_i, dm), lambda i: (i, 0, 0)),
            pl.BlockSpec((1, t, dm), lambda i: (i, 0, 0)),
            pl.BlockSpec((1, h, t, seq_i), lambda i: (i, 0, 0, 0)),
            pl.BlockSpec((1, 3 * dm), lambda i: (0, 0)),
            pl.BlockSpec(memory_space=pltpu.SMEM),
        ],
        out_specs=[
            pl.BlockSpec((1, vocab, t), lambda i: (i, 0, 0)),
            pl.BlockSpec((1, vocab, t), lambda i: (i, 0, 0)),
            pl.BlockSpec((1, 1, t), lambda i: (i, 0, 0)),
        ],
        out_shape=[
            jax.ShapeDtypeStruct((b, vocab, t), jnp.float32),
            jax.ShapeDtypeStruct((b, vocab, t), jnp.float32),
            jax.ShapeDtypeStruct((b, 1, t), jnp.float32),
        ],
        compiler_params=pltpu.CompilerParams(
            dimension_semantics=("parallel",),
            vmem_limit_bytes=100 * 1024 * 1024),
    )(tok, tar_embedded, gen_t, enc_output, dec_state,
      attn_heads, w_row, scal)
    final = jnp.transpose(final_t, (0, 2, 1))
    ptr = jnp.transpose(ptr_t, (0, 2, 1))
    return final, ptr, pg[:, 0, :]


# speedup vs baseline: 3.2265x; 1.0456x over previous
import jax
import jax.numpy as jnp
from jax.experimental import pallas as pl
from jax.experimental.pallas import tpu as pltpu

_VCHUNK = 2000


def _hilo(a):
    hi = a.astype(jnp.bfloat16)
    lo = (a - hi.astype(jnp.float32)).astype(jnp.bfloat16)
    return hi, lo


def _dot2(lhs_hi, lhs_lo, rhs, dims):
    acc = jax.lax.dot_general(lhs_hi, rhs, dims,
                              preferred_element_type=jnp.float32)
    return acc + jax.lax.dot_general(lhs_lo, rhs, dims,
                                     preferred_element_type=jnp.float32)


def _pointer_body(tokc_ref, tokr_ref, emb_ref, genT_ref, enc_ref, dec_ref,
                  ah_ref, w_ref, scal_ref, finalT_ref, ptrT_ref, pg_ref):
    seq_i = ah_ref.shape[3]
    dm = enc_ref.shape[2]
    vocab = genT_ref.shape[1]

    attn = jnp.mean(ah_ref[0], axis=0)
    attn_hi, attn_lo = _hilo(attn)

    enc_hi, enc_lo = _hilo(enc_ref[0])
    dims_ct = (((0,), (1,)), ((), ()))
    contextT = _dot2(enc_hi, enc_lo, attn_hi, dims_ct)
    contextT += jax.lax.dot_general(enc_hi, attn_lo, dims_ct,
                                    preferred_element_type=jnp.float32)

    w1 = w_ref[0:1, 0:dm]
    w2 = w_ref[0:1, dm:2 * dm]
    w3 = w_ref[0:1, 2 * dm:3 * dm]
    z = (jax.lax.dot_general(w1, contextT, (((1,), (0,)), ((), ())),
                             precision=jax.lax.Precision.HIGHEST,
                             preferred_element_type=jnp.float32)
         + jax.lax.dot_general(w2, dec_ref[0], (((1,), (1,)), ((), ())),
                               precision=jax.lax.Precision.HIGHEST,
                               preferred_element_type=jnp.float32)
         + jax.lax.dot_general(w3, emb_ref[0], (((1,), (1,)), ((), ())),
                               precision=jax.lax.Precision.HIGHEST,
                               preferred_element_type=jnp.float32)
         + scal_ref[0])
    p_gen = jax.nn.sigmoid(z)
    sw = scal_ref[1]
    sb = scal_ref[2]

    tok_col = tokc_ref[0][:, 0:1]
    for k0 in range(0, vocab, _VCHUNK):
        ck = min(_VCHUNK, vocab - k0)
        iota = jax.lax.broadcasted_iota(jnp.int32, (seq_i, ck), 1) + k0
        oh = (tok_col == iota).astype(jnp.bfloat16)
        xk = jax.lax.dot_general(
            oh, attn_hi, (((0,), (1,)), ((), ())),
            preferred_element_type=jnp.float32)
        xk += jax.lax.dot_general(
            oh, attn_lo, (((0,), (1,)), ((), ())),
            preferred_element_type=jnp.float32)
        ptrT_ref[0, k0:k0 + ck, :] = xk

    x = ptrT_ref[0]
    m = jnp.max(x, axis=0, keepdims=True)
    lse = m + jnp.log(jnp.sum(jnp.exp(x - m), axis=0, keepdims=True))
    c = sb - sw * lse
    ptr = sw * x + c
    ptrT_ref[0] = ptr
    finalT_ref[0] = p_gen * (genT_ref[0] - ptr) + ptr
    pg_ref[0] = p_gen


def kernel(inp_tokens, tar_embedded, generator_output, enc_output,
           dec_state, attn_heads, W_pgen, b_pgen, scale_w, scale_b):
    b, t, vocab = generator_output.shape
    _, h, _, seq_i = attn_heads.shape
    dm = enc_output.shape[-1]

    tok32 = inp_tokens.astype(jnp.int32)
    tok_bc = jnp.broadcast_to(tok32[:, :, None], (b, seq_i, 8))
    tok_row = tok32.reshape(b, 1, seq_i)
    gen_t = jnp.transpose(generator_output, (0, 2, 1))
    w_row = W_pgen.reshape(1, 3 * dm)
    scal = jnp.concatenate([
        jnp.reshape(b_pgen, (1,)), jnp.reshape(scale_w, (1,)),
        jnp.reshape(scale_b, (1,))
    ]).astype(jnp.float32)

    final_t, ptr_t, pg = pl.pallas_call(
        _pointer_body,
        grid=(b,),
        in_specs=[
            pl.BlockSpec((1, seq_i, 8), lambda i: (i, 0, 0)),
            pl.BlockSpec((1, 1, seq_i), lambda i: (i, 0, 0)),
            pl.BlockSpec((1, t, dm), lambda i: (i, 0, 0)),
            pl.BlockSpec((1, vocab, t), lambda i: (i, 0, 0)),
            pl.BlockSpec((1, seq_i, dm), lambda i: (i, 0, 0)),
            pl.BlockSpec((1, t, dm), lambda i: (i, 0, 0)),
            pl.BlockSpec((1, h, t, seq_i), lambda i: (i, 0, 0, 0)),
            pl.BlockSpec((1, 3 * dm), lambda i: (0, 0)),
            pl.BlockSpec(memory_space=pltpu.SMEM),
        ],
        out_specs=[
            pl.BlockSpec((1, vocab, t), lambda i: (i, 0, 0)),
            pl.BlockSpec((1, vocab, t), lambda i: (i, 0, 0)),
            pl.BlockSpec((1, 1, t), lambda i: (i, 0, 0)),
        ],
        out_shape=[
            jax.ShapeDtypeStruct((b, vocab, t), jnp.float32),
            jax.ShapeDtypeStruct((b, vocab, t), jnp.float32),
            jax.ShapeDtypeStruct((b, 1, t), jnp.float32),
        ],
        compiler_params=pltpu.CompilerParams(
            dimension_semantics=("parallel",),
            vmem_limit_bytes=100 * 1024 * 1024),
    )(tok_bc, tok_row, tar_embedded, gen_t, enc_output, dec_state,
      attn_heads, w_row, scal)
    final = jnp.transpose(final_t, (0, 2, 1))
    ptr = jnp.transpose(ptr_t, (0, 2, 1))
    return final, ptr, pg[:, 0, :]


# speedup vs baseline: 4.2645x; 1.3217x over previous
import jax
import jax.numpy as jnp
from jax.experimental import pallas as pl
from jax.experimental.pallas import tpu as pltpu

_VCHUNK = 2000


def _hilo(a):
    hi = a.astype(jnp.bfloat16)
    lo = (a - hi.astype(jnp.float32)).astype(jnp.bfloat16)
    return hi, lo


def _dot2(lhs_hi, lhs_lo, rhs, dims):
    acc = jax.lax.dot_general(lhs_hi, rhs, dims,
                              preferred_element_type=jnp.float32)
    return acc + jax.lax.dot_general(lhs_lo, rhs, dims,
                                     preferred_element_type=jnp.float32)


def _pointer_body(tokc_ref, tokr_ref, emb_ref, genT_ref, enc_ref, dec_ref,
                  ah_ref, w_ref, scal_ref, finalT_ref, ptrT_ref, pg_ref):
    seq_i = ah_ref.shape[3]
    dm = enc_ref.shape[2]
    vocab = genT_ref.shape[1]

    attn = jnp.mean(ah_ref[0], axis=0)
    attn_hi, attn_lo = _hilo(attn)

    enc_hi, enc_lo = _hilo(enc_ref[0])
    dims_ct = (((0,), (1,)), ((), ()))
    contextT = _dot2(enc_hi, enc_lo, attn_hi, dims_ct)
    contextT += jax.lax.dot_general(enc_hi, attn_lo, dims_ct,
                                    preferred_element_type=jnp.float32)

    w1 = w_ref[0:1, 0:dm]
    w2 = w_ref[0:1, dm:2 * dm]
    w3 = w_ref[0:1, 2 * dm:3 * dm]
    z = (jax.lax.dot_general(w1, contextT, (((1,), (0,)), ((), ())),
                             precision=jax.lax.Precision.HIGHEST,
                             preferred_element_type=jnp.float32)
         + jax.lax.dot_general(w2, dec_ref[0], (((1,), (1,)), ((), ())),
                               precision=jax.lax.Precision.HIGHEST,
                               preferred_element_type=jnp.float32)
         + jax.lax.dot_general(w3, emb_ref[0], (((1,), (1,)), ((), ())),
                               precision=jax.lax.Precision.HIGHEST,
                               preferred_element_type=jnp.float32)
         + scal_ref[0])
    p_gen = jax.nn.sigmoid(z)
    sw = scal_ref[1]
    sb = scal_ref[2]

    tok_col = tokc_ref[0][:, 0:1]
    for k0 in range(0, vocab, _VCHUNK):
        ck = min(_VCHUNK, vocab - k0)
        iota = jax.lax.broadcasted_iota(jnp.int16, (seq_i, ck), 1)
        tokk = (tok_col - k0).astype(jnp.int16)
        oh = jnp.where(tokk == iota, jnp.bfloat16(1.0),
                       jnp.bfloat16(0.0))
        xk = jax.lax.dot_general(
            oh, attn_hi, (((0,), (1,)), ((), ())),
            preferred_element_type=jnp.float32)
        ptrT_ref[0, k0:k0 + ck, :] = xk

    x = ptrT_ref[0]
    m = jnp.max(x, axis=0, keepdims=True)
    lse = m + jnp.log(jnp.sum(jnp.exp(x - m), axis=0, keepdims=True))
    c = sb - sw * lse
    ptr = sw * x + c
    ptrT_ref[0] = ptr
    finalT_ref[0] = p_gen * (genT_ref[0] - ptr) + ptr
    pg_ref[0] = p_gen


def kernel(inp_tokens, tar_embedded, generator_output, enc_output,
           dec_state, attn_heads, W_pgen, b_pgen, scale_w, scale_b):
    b, t, vocab = generator_output.shape
    _, h, _, seq_i = attn_heads.shape
    dm = enc_output.shape[-1]

    tok32 = inp_tokens.astype(jnp.int32)
    tok_bc = jnp.broadcast_to(tok32[:, :, None], (b, seq_i, 8))
    tok_row = tok32.reshape(b, 1, seq_i)
    gen_t = jnp.transpose(generator_output, (0, 2, 1))
    w_row = W_pgen.reshape(1, 3 * dm)
    scal = jnp.concatenate([
        jnp.reshape(b_pgen, (1,)), jnp.reshape(scale_w, (1,)),
        jnp.reshape(scale_b, (1,))
    ]).astype(jnp.float32)

    final_t, ptr_t, pg = pl.pallas_call(
        _pointer_body,
        grid=(b,),
        in_specs=[
            pl.BlockSpec((1, seq_i, 8), lambda i: (i, 0, 0)),
            pl.BlockSpec((1, 1, seq_i), lambda i: (i, 0, 0)),
            pl.BlockSpec((1, t, dm), lambda i: (i, 0, 0)),
            pl.BlockSpec((1, vocab, t), lambda i: (i, 0, 0)),
            pl.BlockSpec((1, seq_i, dm), lambda i: (i, 0, 0)),
            pl.BlockSpec((1, t, dm), lambda i: (i, 0, 0)),
            pl.BlockSpec((1, h, t, seq_i), lambda i: (i, 0, 0, 0)),
            pl.BlockSpec((1, 3 * dm), lambda i: (0, 0)),
            pl.BlockSpec(memory_space=pltpu.SMEM),
        ],
        out_specs=[
            pl.BlockSpec((1, vocab, t), lambda i: (i, 0, 0)),
            pl.BlockSpec((1, vocab, t), lambda i: (i, 0, 0)),
            pl.BlockSpec((1, 1, t), lambda i: (i, 0, 0)),
        ],
        out_shape=[
            jax.ShapeDtypeStruct((b, vocab, t), jnp.float32),
            jax.ShapeDtypeStruct((b, vocab, t), jnp.float32),
            jax.ShapeDtypeStruct((b, 1, t), jnp.float32),
        ],
        compiler_params=pltpu.CompilerParams(
            dimension_semantics=("parallel",),
            vmem_limit_bytes=100 * 1024 * 1024),
    )(tok_bc, tok_row, tar_embedded, gen_t, enc_output, dec_state,
      attn_heads, w_row, scal)
    final = jnp.transpose(final_t, (0, 2, 1))
    ptr = jnp.transpose(ptr_t, (0, 2, 1))
    return final, ptr, pg[:, 0, :]


# speedup vs baseline: 5.3887x; 1.2636x over previous
import jax
import jax.numpy as jnp
from jax.experimental import pallas as pl
from jax.experimental.pallas import tpu as pltpu

_VCHUNK = 2000


def _hilo(a):
    hi = a.astype(jnp.bfloat16)
    lo = (a - hi.astype(jnp.float32)).astype(jnp.bfloat16)
    return hi, lo


def _dot2(lhs_hi, lhs_lo, rhs, dims):
    acc = jax.lax.dot_general(lhs_hi, rhs, dims,
                              preferred_element_type=jnp.float32)
    return acc + jax.lax.dot_general(lhs_lo, rhs, dims,
                                     preferred_element_type=jnp.float32)


def _pointer_body(tokc_ref, tokr_ref, emb_ref, genT_ref, enc_ref, dec_ref,
                  ah_ref, w_ref, scal_ref, finalT_ref, ptrT_ref, pg_ref):
    seq_i = ah_ref.shape[3]
    dm = enc_ref.shape[2]
    vocab = genT_ref.shape[1]

    attn = jnp.mean(ah_ref[0], axis=0)
    attn_hi, attn_lo = _hilo(attn)

    enc_hi, enc_lo = _hilo(enc_ref[0])
    dims_ct = (((0,), (1,)), ((), ()))
    contextT = _dot2(enc_hi, enc_lo, attn_hi, dims_ct)
    contextT += jax.lax.dot_general(enc_hi, attn_lo, dims_ct,
                                    preferred_element_type=jnp.float32)

    w1 = w_ref[0:1, 0:dm]
    w2 = w_ref[0:1, dm:2 * dm]
    w3 = w_ref[0:1, 2 * dm:3 * dm]
    z = (jax.lax.dot_general(w1, contextT, (((1,), (0,)), ((), ())),
                             precision=jax.lax.Precision.HIGHEST,
                             preferred_element_type=jnp.float32)
         + jax.lax.dot_general(w2, dec_ref[0], (((1,), (1,)), ((), ())),
                               precision=jax.lax.Precision.HIGHEST,
                               preferred_element_type=jnp.float32)
         + jax.lax.dot_general(w3, emb_ref[0], (((1,), (1,)), ((), ())),
                               precision=jax.lax.Precision.HIGHEST,
                               preferred_element_type=jnp.float32)
         + scal_ref[0])
    p_gen = jax.nn.sigmoid(z)
    sw = scal_ref[1]
    sb = scal_ref[2]

    tok_row = tokr_ref[0]
    for k0 in range(0, vocab, _VCHUNK):
        ck = min(_VCHUNK, vocab - k0)
        iota = jax.lax.broadcasted_iota(jnp.int16, (ck, seq_i), 0)
        tokk = (tok_row - k0).astype(jnp.int16)
        oh = jnp.where(iota == tokk, jnp.bfloat16(1.0),
                       jnp.bfloat16(0.0))
        xk = jax.lax.dot_general(
            oh, attn_hi, (((1,), (1,)), ((), ())),
            preferred_element_type=jnp.float32)
        ptrT_ref[0, k0:k0 + ck, :] = xk

    x = ptrT_ref[0]
    m = jnp.max(x, axis=0, keepdims=True)
    lse = m + jnp.log(jnp.sum(jnp.exp(x - m), axis=0, keepdims=True))
    c = sb - sw * lse
    ptr = sw * x + c
    ptrT_ref[0] = ptr
    finalT_ref[0] = p_gen * (genT_ref[0] - ptr) + ptr
    pg_ref[0] = p_gen


def kernel(inp_tokens, tar_embedded, generator_output, enc_output,
           dec_state, attn_heads, W_pgen, b_pgen, scale_w, scale_b):
    b, t, vocab = generator_output.shape
    _, h, _, seq_i = attn_heads.shape
    dm = enc_output.shape[-1]

    tok32 = inp_tokens.astype(jnp.int32)
    tok_bc = jnp.broadcast_to(tok32[:, :, None], (b, seq_i, 8))
    tok_row = tok32.reshape(b, 1, seq_i)
    gen_t = jnp.transpose(generator_output, (0, 2, 1))
    w_row = W_pgen.reshape(1, 3 * dm)
    scal = jnp.concatenate([
        jnp.reshape(b_pgen, (1,)), jnp.reshape(scale_w, (1,)),
        jnp.reshape(scale_b, (1,))
    ]).astype(jnp.float32)

    final_t, ptr_t, pg = pl.pallas_call(
        _pointer_body,
        grid=(b,),
        in_specs=[
            pl.BlockSpec((1, seq_i, 8), lambda i: (i, 0, 0)),
            pl.BlockSpec((1, 1, seq_i), lambda i: (i, 0, 0)),
            pl.BlockSpec((1, t, dm), lambda i: (i, 0, 0)),
            pl.BlockSpec((1, vocab, t), lambda i: (i, 0, 0)),
            pl.BlockSpec((1, seq_i, dm), lambda i: (i, 0, 0)),
            pl.BlockSpec((1, t, dm), lambda i: (i, 0, 0)),
            pl.BlockSpec((1, h, t, seq_i), lambda i: (i, 0, 0, 0)),
            pl.BlockSpec((1, 3 * dm), lambda i: (0, 0)),
            pl.BlockSpec(memory_space=pltpu.SMEM),
        ],
        out_specs=[
            pl.BlockSpec((1, vocab, t), lambda i: (i, 0, 0)),
            pl.BlockSpec((1, vocab, t), lambda i: (i, 0, 0)),
            pl.BlockSpec((1, 1, t), lambda i: (i, 0, 0)),
        ],
        out_shape=[
            jax.ShapeDtypeStruct((b, vocab, t), jnp.float32),
            jax.ShapeDtypeStruct((b, vocab, t), jnp.float32),
            jax.ShapeDtypeStruct((b, 1, t), jnp.float32),
        ],
        compiler_params=pltpu.CompilerParams(
            dimension_semantics=("parallel",),
            vmem_limit_bytes=100 * 1024 * 1024),
    )(tok_bc, tok_row, tar_embedded, gen_t, enc_output, dec_state,
      attn_heads, w_row, scal)
    final = jnp.transpose(final_t, (0, 2, 1))
    ptr = jnp.transpose(ptr_t, (0, 2, 1))
    return final, ptr, pg[:, 0, :]
